# Initial kernel scaffold; baseline (speedup 1.0000x reference)
import jax
import jax.numpy as jnp
from jax.experimental import pallas as pl


def kernel(xyz1, xyz2, points1, points2, mlp0_w, mlp0_b, mlp1_w, mlp1_b, wn1_w0, wn1_b0, wn1_w1, wn1_b1, wn1_w2, wn1_b2, wn2_w0, wn2_b0, wn2_w1, wn2_b1, wn2_w2, wn2_b2):
    raise NotImplementedError("write your pallas kernel here")


# speedup vs baseline: 19.3863x; 19.3863x over previous
import functools

import jax
import jax.numpy as jnp
from jax import lax
from jax.experimental import pallas as pl
from jax.experimental.pallas import tpu as pltpu
from jax.experimental.pallas import tpu_sc as plsc

N = 8192
D = 128
K = 8
BQ = 128
GB = BQ * K
F32 = jnp.float32
FBIG = 3e38
IBIG = 2**30


def _dot(a, b):
    return jnp.dot(a, b, preferred_element_type=F32,
                   precision=jax.lax.Precision.HIGHEST)


def _leaky(x):
    return jnp.where(x >= 0, x, 0.1 * x)


def _wnet(x, w0, b0, w1, b1, w2, b2):
    x = jax.nn.relu(_dot(x, w0) + b0)
    x = jax.nn.relu(_dot(x, w1) + b1)
    x = jax.nn.relu(_dot(x, w2) + b2)
    return x


def _top8(d, iota, out_cols):
    acc = jnp.zeros((BQ, K), jnp.int32)
    for s in range(K):
        m = jnp.min(d, axis=1, keepdims=True)
        j = jnp.min(jnp.where(d <= m, iota, IBIG), axis=1)
        acc = jnp.where(out_cols == s, j[:, None], acc)
        d = jnp.where(iota == j[:, None], FBIG, d)
    return acc


def _knn_body(q_ref, k2_ref, k1_ref, idx1_ref, idx2_ref):
    q = q_ref[...]
    k2 = k2_ref[...]
    k1 = k1_ref[...]
    qn = jnp.sum(q * q, axis=1, keepdims=True)
    iota = lax.broadcasted_iota(jnp.int32, (BQ, N), 1)
    out_cols = lax.broadcasted_iota(jnp.int32, (BQ, K), 1)

    kn2 = jnp.sum(k2 * k2, axis=0, keepdims=True)
    d1 = qn + kn2 - 2.0 * jnp.dot(q, k2, preferred_element_type=F32)
    idx1_ref[...] = _top8(d1, iota, out_cols)

    kn1 = jnp.sum(k1 * k1, axis=0, keepdims=True)
    d2 = qn + kn1 - 2.0 * jnp.dot(q, k1, preferred_element_type=F32)
    idx2_ref[...] = _top8(d2, iota, out_cols)


def _knn(x1r, x2t, x1t):
    return pl.pallas_call(
        _knn_body,
        grid=(N // BQ,),
        in_specs=[
            pl.BlockSpec((BQ, 3), lambda i: (i, 0)),
            pl.BlockSpec((3, N), lambda i: (0, 0)),
            pl.BlockSpec((3, N), lambda i: (0, 0)),
        ],
        out_specs=[
            pl.BlockSpec((BQ, K), lambda i: (i, 0)),
            pl.BlockSpec((BQ, K), lambda i: (i, 0)),
        ],
        out_shape=[
            jax.ShapeDtypeStruct((N, K), jnp.int32),
            jax.ShapeDtypeStruct((N, K), jnp.int32),
        ],
    )(x1r, x2t, x1t)


def _proj_body(p1_ref, p2_ref, wa_ref, wb_ref, t1_ref, t2_ref):
    t1_ref[...] = _dot(p1_ref[...], wa_ref[...])
    t2_ref[...] = _dot(p2_ref[...], wb_ref[...])


def _proj(p1r, p2r, wa, wb):
    bm = 512
    return pl.pallas_call(
        _proj_body,
        grid=(N // bm,),
        in_specs=[
            pl.BlockSpec((bm, D), lambda i: (i, 0)),
            pl.BlockSpec((bm, D), lambda i: (i, 0)),
            pl.BlockSpec((D, D), lambda i: (0, 0)),
            pl.BlockSpec((D, D), lambda i: (0, 0)),
        ],
        out_specs=[
            pl.BlockSpec((bm, D), lambda i: (i, 0)),
            pl.BlockSpec((bm, D), lambda i: (i, 0)),
        ],
        out_shape=[
            jax.ShapeDtypeStruct((N, D), F32),
            jax.ShapeDtypeStruct((N, D), F32),
        ],
    )(p1r, p2r, wa, wb)


_NC = 2
_NS = 16
_NW = _NC * _NS
_CH = 128


def _sc_gather2_body(tab_a_hbm, idx_a_hbm, tab_b_hbm, idx_b_hbm,
                     out_a_hbm, out_b_hbm,
                     idx_v, rows_a, rows_b, sem):
    wid = lax.axis_index("s") * _NC + lax.axis_index("c")
    nk = idx_a_hbm.shape[0] * idx_a_hbm.shape[1]
    rows_per_w = nk // _NW
    chunks = rows_per_w // _CH
    crow0 = wid * chunks
    pltpu.sync_copy(idx_a_hbm.at[pl.ds(crow0, chunks)], idx_v)
    for c in range(chunks):
        base = (crow0 + c) * _CH
        pltpu.async_copy(tab_a_hbm.at[idx_v.at[c]], rows_a, sem).wait()
        pltpu.sync_copy(rows_a, out_a_hbm.at[pl.ds(base, _CH)])
    pltpu.sync_copy(idx_b_hbm.at[pl.ds(crow0, chunks)], idx_v)
    for c in range(chunks):
        base = (crow0 + c) * _CH
        pltpu.async_copy(tab_b_hbm.at[idx_v.at[c]], rows_b, sem).wait()
        pltpu.sync_copy(rows_b, out_b_hbm.at[pl.ds(base, _CH)])


def _sc_gather2(tab_a, idx_a, tab_b, idx_b):
    nk = idx_a.shape[0] * idx_a.shape[1]
    da, db = tab_a.shape[1], tab_b.shape[1]
    chunks = nk // _NW // _CH
    mesh = plsc.VectorSubcoreMesh(core_axis_name="c", subcore_axis_name="s")
    f = pl.kernel(
        _sc_gather2_body,
        out_type=[
            jax.ShapeDtypeStruct((nk, da), F32),
            jax.ShapeDtypeStruct((nk, db), F32),
        ],
        mesh=mesh,
        scratch_types=[
            pltpu.VMEM((chunks, _CH), jnp.int32),
            pltpu.VMEM((_CH, da), F32),
            pltpu.VMEM((_CH, db), F32),
            pltpu.SemaphoreType.DMA,
        ],
        compiler_params=pltpu.CompilerParams(use_tc_tiling_on_sc=False),
    )
    return f(tab_a, idx_a, tab_b, idx_b)


def _sc_gather1_body(tab_hbm, idx_hbm, out_hbm, idx_v, rows_v, sem):
    wid = lax.axis_index("s") * _NC + lax.axis_index("c")
    nk = idx_hbm.shape[0] * idx_hbm.shape[1]
    chunks = nk // _NW // _CH
    crow0 = wid * chunks
    pltpu.sync_copy(idx_hbm.at[pl.ds(crow0, chunks)], idx_v)
    for c in range(chunks):
        base = (crow0 + c) * _CH
        pltpu.async_copy(tab_hbm.at[idx_v.at[c]], rows_v, sem).wait()
        pltpu.sync_copy(rows_v, out_hbm.at[pl.ds(base, _CH)])


def _sc_gather1(tab, idx):
    nk = idx.shape[0] * idx.shape[1]
    d = tab.shape[1]
    chunks = nk // _NW // _CH
    mesh = plsc.VectorSubcoreMesh(core_axis_name="c", subcore_axis_name="s")
    f = pl.kernel(
        _sc_gather1_body,
        out_type=jax.ShapeDtypeStruct((nk, d), F32),
        mesh=mesh,
        scratch_types=[
            pltpu.VMEM((chunks, _CH), jnp.int32),
            pltpu.VMEM((_CH, d), F32),
            pltpu.SemaphoreType.DMA,
        ],
        compiler_params=pltpu.CompilerParams(use_tc_tiling_on_sc=False),
    )
    return f(tab, idx)


def _rep_mats():
    r_rows = lax.broadcasted_iota(jnp.int32, (GB, BQ), 0)
    r_cols = lax.broadcasted_iota(jnp.int32, (GB, BQ), 1)
    R = (r_rows // K == r_cols).astype(F32)
    return R


def _stage1_body(g_ref, t1_ref, x1_ref, wc_ref, b0_ref, m1_ref, b1_ref,
                 w0_ref, c0_ref, w1_ref, c1_ref, w2_ref, c2_ref, out_ref):
    g = g_ref[...]
    R = _rep_mats()
    t1rep = _dot(R, t1_ref[...])
    x1rep = _dot(R, x1_ref[...])
    dxyz = g[:, D:D + 3] - x1rep
    h = g[:, 0:D] + t1rep + _dot(dxyz, wc_ref[...]) + b0_ref[...]
    h = _leaky(h)
    y = _leaky(_dot(h, m1_ref[...]) + b1_ref[...])
    w = _wnet(dxyz, w0_ref[...], c0_ref[...], w1_ref[...], c1_ref[...],
              w2_ref[...], c2_ref[...])
    prod = w * y
    out_ref[...] = _dot(R.T, prod)


def _stage1(g, t1, x1r, wc, b0, m1w, b1, w0, c0, w1, c1, w2, c2):
    dg = g.shape[1]
    return pl.pallas_call(
        _stage1_body,
        grid=(N // BQ,),
        in_specs=[
            pl.BlockSpec((GB, dg), lambda i: (i, 0)),
            pl.BlockSpec((BQ, D), lambda i: (i, 0)),
            pl.BlockSpec((BQ, 3), lambda i: (i, 0)),
            pl.BlockSpec((3, D), lambda i: (0, 0)),
            pl.BlockSpec((1, D), lambda i: (0, 0)),
            pl.BlockSpec((D, 64), lambda i: (0, 0)),
            pl.BlockSpec((1, 64), lambda i: (0, 0)),
            pl.BlockSpec((3, 8), lambda i: (0, 0)),
            pl.BlockSpec((1, 8), lambda i: (0, 0)),
            pl.BlockSpec((8, 8), lambda i: (0, 0)),
            pl.BlockSpec((1, 8), lambda i: (0, 0)),
            pl.BlockSpec((8, 64), lambda i: (0, 0)),
            pl.BlockSpec((1, 64), lambda i: (0, 0)),
        ],
        out_specs=pl.BlockSpec((BQ, 64), lambda i: (i, 0)),
        out_shape=jax.ShapeDtypeStruct((N, 64), F32),
    )(g, t1, x1r, wc, b0, m1w, b1, w0, c0, w1, c1, w2, c2)


def _stage2_body(q_ref, g3_ref, x1_ref, w0_ref, c0_ref, w1_ref, c1_ref,
                 w2_ref, c2_ref, out_ref):
    R = _rep_mats()
    x1rep = _dot(R, x1_ref[...])
    dxyz2 = q_ref[...][:, 0:3] - x1rep
    w = _wnet(dxyz2, w0_ref[...], c0_ref[...], w1_ref[...], c1_ref[...],
              w2_ref[...], c2_ref[...])
    prod = w * g3_ref[...]
    out_ref[...] = _dot(R.T, prod)


def _stage2(qg, g3, x1r, w0, c0, w1, c1, w2, c2):
    dq = qg.shape[1]
    return pl.pallas_call(
        _stage2_body,
        grid=(N // BQ,),
        in_specs=[
            pl.BlockSpec((GB, dq), lambda i: (i, 0)),
            pl.BlockSpec((GB, 64), lambda i: (i, 0)),
            pl.BlockSpec((BQ, 3), lambda i: (i, 0)),
            pl.BlockSpec((3, 8), lambda i: (0, 0)),
            pl.BlockSpec((1, 8), lambda i: (0, 0)),
            pl.BlockSpec((8, 8), lambda i: (0, 0)),
            pl.BlockSpec((1, 8), lambda i: (0, 0)),
            pl.BlockSpec((8, 64), lambda i: (0, 0)),
            pl.BlockSpec((1, 64), lambda i: (0, 0)),
        ],
        out_specs=pl.BlockSpec((BQ, 64), lambda i: (i, 0)),
        out_shape=jax.ShapeDtypeStruct((N, 64), F32),
    )(qg, g3, x1r, w0, c0, w1, c1, w2, c2)


def kernel(xyz1, xyz2, points1, points2, mlp0_w, mlp0_b, mlp1_w, mlp1_b,
           wn1_w0, wn1_b0, wn1_w1, wn1_b1, wn1_w2, wn1_b2,
           wn2_w0, wn2_b0, wn2_w1, wn2_b1, wn2_w2, wn2_b2):
    x1t = xyz1[0]
    x2t = xyz2[0]
    x1r = x1t.T
    x2r = x2t.T
    p1r = points1[0].T
    p2r = points2[0].T

    wa = mlp0_w[0:D]
    wb = mlp0_w[D:2 * D]
    wc = mlp0_w[2 * D:2 * D + 3]

    idx1, idx2 = _knn(x1r, x2t, x1t)
    t1, t2 = _proj(p1r, p2r, wa, wb)

    pad13 = jnp.zeros((N, 13), F32)
    tab2 = jnp.concatenate([t2, x2r, pad13], axis=1)
    tabq = jnp.concatenate([x1r, pad13], axis=1)

    idx1_2d = idx1.reshape(N * K // _CH, _CH)
    idx2_2d = idx2.reshape(N * K // _CH, _CH)

    g, qg = _sc_gather2(tab2, idx1_2d, tabq, idx2_2d)

    p2p = _stage1(g, t1, x1r, wc,
                  mlp0_b.reshape(1, D), mlp1_w, mlp1_b.reshape(1, 64),
                  wn1_w0, wn1_b0.reshape(1, 8), wn1_w1, wn1_b1.reshape(1, 8),
                  wn1_w2, wn1_b2.reshape(1, 64))

    g3 = _sc_gather1(p2p, idx2_2d)

    patch = _stage2(qg, g3, x1r,
                    wn2_w0, wn2_b0.reshape(1, 8), wn2_w1, wn2_b1.reshape(1, 8),
                    wn2_w2, wn2_b2.reshape(1, 64))

    return jnp.transpose(patch, (1, 0))[None]


# speedup vs baseline: 23.3038x; 1.2021x over previous
import functools

import jax
import jax.numpy as jnp
from jax import lax
from jax.experimental import pallas as pl
from jax.experimental.pallas import tpu as pltpu
from jax.experimental.pallas import tpu_sc as plsc

N = 8192
D = 128
K = 8
BQ = 128
GB = BQ * K
F32 = jnp.float32
FBIG = 3e38
IBIG = 2**30


def _dot(a, b):
    return jnp.dot(a, b, preferred_element_type=F32,
                   precision=jax.lax.Precision.HIGHEST)


def _leaky(x):
    return jnp.where(x >= 0, x, 0.1 * x)


def _wnet(x, w0, b0, w1, b1, w2, b2):
    x = jax.nn.relu(_dot(x, w0) + b0)
    x = jax.nn.relu(_dot(x, w1) + b1)
    x = jax.nn.relu(_dot(x, w2) + b2)
    return x


def _top8(d, iota, out_cols):
    acc = jnp.zeros((BQ, K), jnp.int32)
    for s in range(K):
        m = jnp.min(d, axis=1, keepdims=True)
        eq = d <= m
        j = jnp.min(jnp.where(eq, iota, IBIG), axis=1)
        acc = jnp.where(out_cols == s, j[:, None], acc)
        d = jnp.where(eq, FBIG, d)
    return acc


def _knn_body(q_ref, k2_ref, k1_ref, idx1_ref, idx2_ref):
    q = q_ref[...]
    k2 = k2_ref[...]
    k1 = k1_ref[...]
    qn = jnp.sum(q * q, axis=1, keepdims=True)
    iota = lax.broadcasted_iota(jnp.int32, (BQ, N), 1)
    out_cols = lax.broadcasted_iota(jnp.int32, (BQ, K), 1)

    kn2 = jnp.sum(k2 * k2, axis=0, keepdims=True)
    d1 = qn + kn2 - 2.0 * jnp.dot(q, k2, preferred_element_type=F32)
    idx1_ref[...] = _top8(d1, iota, out_cols)

    kn1 = jnp.sum(k1 * k1, axis=0, keepdims=True)
    d2 = qn + kn1 - 2.0 * jnp.dot(q, k1, preferred_element_type=F32)
    idx2_ref[...] = _top8(d2, iota, out_cols)


def _knn(x1r, x2t, x1t):
    return pl.pallas_call(
        _knn_body,
        grid=(N // BQ,),
        in_specs=[
            pl.BlockSpec((BQ, 3), lambda i: (i, 0)),
            pl.BlockSpec((3, N), lambda i: (0, 0)),
            pl.BlockSpec((3, N), lambda i: (0, 0)),
        ],
        out_specs=[
            pl.BlockSpec((BQ, K), lambda i: (i, 0)),
            pl.BlockSpec((BQ, K), lambda i: (i, 0)),
        ],
        out_shape=[
            jax.ShapeDtypeStruct((N, K), jnp.int32),
            jax.ShapeDtypeStruct((N, K), jnp.int32),
        ],
    )(x1r, x2t, x1t)


def _proj_body(p1_ref, p2_ref, wa_ref, wb_ref, t1_ref, t2_ref):
    t1_ref[...] = _dot(p1_ref[...], wa_ref[...])
    t2_ref[...] = _dot(p2_ref[...], wb_ref[...])


def _proj(p1r, p2r, wa, wb):
    bm = 512
    return pl.pallas_call(
        _proj_body,
        grid=(N // bm,),
        in_specs=[
            pl.BlockSpec((bm, D), lambda i: (i, 0)),
            pl.BlockSpec((bm, D), lambda i: (i, 0)),
            pl.BlockSpec((D, D), lambda i: (0, 0)),
            pl.BlockSpec((D, D), lambda i: (0, 0)),
        ],
        out_specs=[
            pl.BlockSpec((bm, D), lambda i: (i, 0)),
            pl.BlockSpec((bm, D), lambda i: (i, 0)),
        ],
        out_shape=[
            jax.ShapeDtypeStruct((N, D), F32),
            jax.ShapeDtypeStruct((N, D), F32),
        ],
    )(p1r, p2r, wa, wb)


_NC = 2
_NS = 16
_NW = _NC * _NS
_CH = 128


def _sc_gather2_body(tab_a_hbm, idx_a_hbm, tab_b_hbm, idx_b_hbm,
                     out_a_hbm, out_b_hbm,
                     idx_v, rows_a, rows_b, sem):
    wid = lax.axis_index("s") * _NC + lax.axis_index("c")
    nk = idx_a_hbm.shape[0] * idx_a_hbm.shape[1]
    rows_per_w = nk // _NW
    chunks = rows_per_w // _CH
    crow0 = wid * chunks
    pltpu.sync_copy(idx_a_hbm.at[pl.ds(crow0, chunks)], idx_v)
    for c in range(chunks):
        base = (crow0 + c) * _CH
        pltpu.async_copy(tab_a_hbm.at[idx_v.at[c]], rows_a, sem).wait()
        pltpu.sync_copy(rows_a, out_a_hbm.at[pl.ds(base, _CH)])
    pltpu.sync_copy(idx_b_hbm.at[pl.ds(crow0, chunks)], idx_v)
    for c in range(chunks):
        base = (crow0 + c) * _CH
        pltpu.async_copy(tab_b_hbm.at[idx_v.at[c]], rows_b, sem).wait()
        pltpu.sync_copy(rows_b, out_b_hbm.at[pl.ds(base, _CH)])


def _sc_gather2(tab_a, idx_a, tab_b, idx_b):
    nk = idx_a.shape[0] * idx_a.shape[1]
    da, db = tab_a.shape[1], tab_b.shape[1]
    chunks = nk // _NW // _CH
    mesh = plsc.VectorSubcoreMesh(core_axis_name="c", subcore_axis_name="s")
    f = pl.kernel(
        _sc_gather2_body,
        out_type=[
            jax.ShapeDtypeStruct((nk, da), F32),
            jax.ShapeDtypeStruct((nk, db), F32),
        ],
        mesh=mesh,
        scratch_types=[
            pltpu.VMEM((chunks, _CH), jnp.int32),
            pltpu.VMEM((_CH, da), F32),
            pltpu.VMEM((_CH, db), F32),
            pltpu.SemaphoreType.DMA,
        ],
        compiler_params=pltpu.CompilerParams(use_tc_tiling_on_sc=False),
    )
    return f(tab_a, idx_a, tab_b, idx_b)


def _sc_gather1_body(tab_hbm, idx_hbm, out_hbm, idx_v, rows_v, sem):
    wid = lax.axis_index("s") * _NC + lax.axis_index("c")
    nk = idx_hbm.shape[0] * idx_hbm.shape[1]
    chunks = nk // _NW // _CH
    crow0 = wid * chunks
    pltpu.sync_copy(idx_hbm.at[pl.ds(crow0, chunks)], idx_v)
    for c in range(chunks):
        base = (crow0 + c) * _CH
        pltpu.async_copy(tab_hbm.at[idx_v.at[c]], rows_v, sem).wait()
        pltpu.sync_copy(rows_v, out_hbm.at[pl.ds(base, _CH)])


def _sc_gather1(tab, idx):
    nk = idx.shape[0] * idx.shape[1]
    d = tab.shape[1]
    chunks = nk // _NW // _CH
    mesh = plsc.VectorSubcoreMesh(core_axis_name="c", subcore_axis_name="s")
    f = pl.kernel(
        _sc_gather1_body,
        out_type=jax.ShapeDtypeStruct((nk, d), F32),
        mesh=mesh,
        scratch_types=[
            pltpu.VMEM((chunks, _CH), jnp.int32),
            pltpu.VMEM((_CH, d), F32),
            pltpu.SemaphoreType.DMA,
        ],
        compiler_params=pltpu.CompilerParams(use_tc_tiling_on_sc=False),
    )
    return f(tab, idx)


def _rep8(x):
    c = x.shape[1]
    return jnp.broadcast_to(x[:, None, :], (BQ, K, c)).reshape(GB, c)


def _sum8(x):
    c = x.shape[1]
    return x.reshape(BQ, K, c).sum(axis=1)


def _stage1_body(g_ref, t1_ref, x1_ref, wc_ref, b0_ref, m1_ref, b1_ref,
                 w0_ref, c0_ref, w1_ref, c1_ref, w2_ref, c2_ref, out_ref):
    g = g_ref[...]
    t1rep = _rep8(t1_ref[...])
    x1rep = _rep8(x1_ref[...])
    dxyz = g[:, D:D + 3] - x1rep
    h = g[:, 0:D] + t1rep + _dot(dxyz, wc_ref[...]) + b0_ref[...]
    h = _leaky(h)
    y = _leaky(_dot(h, m1_ref[...]) + b1_ref[...])
    w = _wnet(dxyz, w0_ref[...], c0_ref[...], w1_ref[...], c1_ref[...],
              w2_ref[...], c2_ref[...])
    prod = w * y
    out_ref[...] = _sum8(prod)


def _stage1(g, t1, x1r, wc, b0, m1w, b1, w0, c0, w1, c1, w2, c2):
    dg = g.shape[1]
    return pl.pallas_call(
        _stage1_body,
        grid=(N // BQ,),
        in_specs=[
            pl.BlockSpec((GB, dg), lambda i: (i, 0)),
            pl.BlockSpec((BQ, D), lambda i: (i, 0)),
            pl.BlockSpec((BQ, 3), lambda i: (i, 0)),
            pl.BlockSpec((3, D), lambda i: (0, 0)),
            pl.BlockSpec((1, D), lambda i: (0, 0)),
            pl.BlockSpec((D, 64), lambda i: (0, 0)),
            pl.BlockSpec((1, 64), lambda i: (0, 0)),
            pl.BlockSpec((3, 8), lambda i: (0, 0)),
            pl.BlockSpec((1, 8), lambda i: (0, 0)),
            pl.BlockSpec((8, 8), lambda i: (0, 0)),
            pl.BlockSpec((1, 8), lambda i: (0, 0)),
            pl.BlockSpec((8, 64), lambda i: (0, 0)),
            pl.BlockSpec((1, 64), lambda i: (0, 0)),
        ],
        out_specs=pl.BlockSpec((BQ, 64), lambda i: (i, 0)),
        out_shape=jax.ShapeDtypeStruct((N, 64), F32),
    )(g, t1, x1r, wc, b0, m1w, b1, w0, c0, w1, c1, w2, c2)


def _stage2_body(q_ref, g3_ref, x1_ref, w0_ref, c0_ref, w1_ref, c1_ref,
                 w2_ref, c2_ref, out_ref):
    x1rep = _rep8(x1_ref[...])
    dxyz2 = q_ref[...][:, 0:3] - x1rep
    w = _wnet(dxyz2, w0_ref[...], c0_ref[...], w1_ref[...], c1_ref[...],
              w2_ref[...], c2_ref[...])
    prod = w * g3_ref[...]
    out_ref[...] = _sum8(prod)


def _stage2(qg, g3, x1r, w0, c0, w1, c1, w2, c2):
    dq = qg.shape[1]
    return pl.pallas_call(
        _stage2_body,
        grid=(N // BQ,),
        in_specs=[
            pl.BlockSpec((GB, dq), lambda i: (i, 0)),
            pl.BlockSpec((GB, 64), lambda i: (i, 0)),
            pl.BlockSpec((BQ, 3), lambda i: (i, 0)),
            pl.BlockSpec((3, 8), lambda i: (0, 0)),
            pl.BlockSpec((1, 8), lambda i: (0, 0)),
            pl.BlockSpec((8, 8), lambda i: (0, 0)),
            pl.BlockSpec((1, 8), lambda i: (0, 0)),
            pl.BlockSpec((8, 64), lambda i: (0, 0)),
            pl.BlockSpec((1, 64), lambda i: (0, 0)),
        ],
        out_specs=pl.BlockSpec((BQ, 64), lambda i: (i, 0)),
        out_shape=jax.ShapeDtypeStruct((N, 64), F32),
    )(qg, g3, x1r, w0, c0, w1, c1, w2, c2)


def kernel(xyz1, xyz2, points1, points2, mlp0_w, mlp0_b, mlp1_w, mlp1_b,
           wn1_w0, wn1_b0, wn1_w1, wn1_b1, wn1_w2, wn1_b2,
           wn2_w0, wn2_b0, wn2_w1, wn2_b1, wn2_w2, wn2_b2):
    x1t = xyz1[0]
    x2t = xyz2[0]
    x1r = x1t.T
    x2r = x2t.T
    p1r = points1[0].T
    p2r = points2[0].T

    wa = mlp0_w[0:D]
    wb = mlp0_w[D:2 * D]
    wc = mlp0_w[2 * D:2 * D + 3]

    idx1, idx2 = _knn(x1r, x2t, x1t)
    t1, t2 = _proj(p1r, p2r, wa, wb)

    pad13 = jnp.zeros((N, 13), F32)
    tab2 = jnp.concatenate([t2, x2r, pad13], axis=1)
    tabq = jnp.concatenate([x1r, pad13], axis=1)

    idx1_2d = idx1.reshape(N * K // _CH, _CH)
    idx2_2d = idx2.reshape(N * K // _CH, _CH)

    g, qg = _sc_gather2(tab2, idx1_2d, tabq, idx2_2d)

    p2p = _stage1(g, t1, x1r, wc,
                  mlp0_b.reshape(1, D), mlp1_w, mlp1_b.reshape(1, 64),
                  wn1_w0, wn1_b0.reshape(1, 8), wn1_w1, wn1_b1.reshape(1, 8),
                  wn1_w2, wn1_b2.reshape(1, 64))

    g3 = _sc_gather1(p2p, idx2_2d)

    patch = _stage2(qg, g3, x1r,
                    wn2_w0, wn2_b0.reshape(1, 8), wn2_w1, wn2_b1.reshape(1, 8),
                    wn2_w2, wn2_b2.reshape(1, 64))

    return jnp.transpose(patch, (1, 0))[None]


# speedup vs baseline: 23.3880x; 1.0036x over previous
import functools

import jax
import jax.numpy as jnp
from jax import lax
from jax.experimental import pallas as pl
from jax.experimental.pallas import tpu as pltpu
from jax.experimental.pallas import tpu_sc as plsc

N = 8192
D = 128
K = 8
BQ = 128
GB = BQ * K
F32 = jnp.float32


def _dot(a, b):
    return jnp.dot(a, b, preferred_element_type=F32,
                   precision=jax.lax.Precision.HIGHEST)


def _leaky(x):
    return jnp.where(x >= 0, x, 0.1 * x)


def _wnet(x, w0, b0, w1, b1, w2, b2):
    x = jax.nn.relu(_dot(x, w0) + b0)
    x = jax.nn.relu(_dot(x, w1) + b1)
    x = jax.nn.relu(_dot(x, w2) + b2)
    return x


def _top8(d, iota, out_cols):
    acc = jnp.zeros((BQ, K), jnp.int32)
    for s in range(K):
        m = jnp.min(d, axis=1, keepdims=True)
        eq = d <= m
        j = jnp.min(jnp.where(eq, iota, 0x7FFFFFFF), axis=1)
        acc = jnp.where(out_cols == s, j[:, None], acc)
        d = jnp.where(eq, 3e38, d)
    return acc


def _knn_body(q_ref, k2_ref, k1_ref, idx1_ref, idx2_ref):
    q = q_ref[...]
    k2 = k2_ref[...]
    k1 = k1_ref[...]
    qn = jnp.sum(q * q, axis=1, keepdims=True)
    iota = lax.broadcasted_iota(jnp.int32, (BQ, N), 1)
    out_cols = lax.broadcasted_iota(jnp.int32, (BQ, K), 1)

    kn2 = jnp.sum(k2 * k2, axis=0, keepdims=True)
    d1 = qn + kn2 - 2.0 * jnp.dot(q, k2, preferred_element_type=F32)
    idx1_ref[...] = _top8(d1, iota, out_cols)

    kn1 = jnp.sum(k1 * k1, axis=0, keepdims=True)
    d2 = qn + kn1 - 2.0 * jnp.dot(q, k1, preferred_element_type=F32)
    idx2_ref[...] = _top8(d2, iota, out_cols)


def _knn(x1r, x2t, x1t):
    return pl.pallas_call(
        _knn_body,
        grid=(N // BQ,),
        in_specs=[
            pl.BlockSpec((BQ, 3), lambda i: (i, 0)),
            pl.BlockSpec((3, N), lambda i: (0, 0)),
            pl.BlockSpec((3, N), lambda i: (0, 0)),
        ],
        out_specs=[
            pl.BlockSpec((BQ, K), lambda i: (i, 0)),
            pl.BlockSpec((BQ, K), lambda i: (i, 0)),
        ],
        out_shape=[
            jax.ShapeDtypeStruct((N, K), jnp.int32),
            jax.ShapeDtypeStruct((N, K), jnp.int32),
        ],
    )(x1r, x2t, x1t)


def _proj_body(p1_ref, p2_ref, wa_ref, wb_ref, t1_ref, t2_ref):
    dn = (((0,), (0,)), ((), ()))
    t1_ref[...] = lax.dot_general(p1_ref[...], wa_ref[...], dn,
                                  preferred_element_type=F32,
                                  precision=jax.lax.Precision.HIGHEST)
    t2_ref[...] = lax.dot_general(p2_ref[...], wb_ref[...], dn,
                                  preferred_element_type=F32,
                                  precision=jax.lax.Precision.HIGHEST)


def _proj(p1t, p2t, wa, wb):
    bm = 512
    return pl.pallas_call(
        _proj_body,
        grid=(N // bm,),
        in_specs=[
            pl.BlockSpec((D, bm), lambda i: (0, i)),
            pl.BlockSpec((D, bm), lambda i: (0, i)),
            pl.BlockSpec((D, D), lambda i: (0, 0)),
            pl.BlockSpec((D, D), lambda i: (0, 0)),
        ],
        out_specs=[
            pl.BlockSpec((bm, D), lambda i: (i, 0)),
            pl.BlockSpec((bm, D), lambda i: (i, 0)),
        ],
        out_shape=[
            jax.ShapeDtypeStruct((N, D), F32),
            jax.ShapeDtypeStruct((N, D), F32),
        ],
    )(p1t, p2t, wa, wb)


_NC = 2
_NS = 16
_NW = _NC * _NS
_CH = 128


def _sc_gather2_body(tab_a_hbm, idx_a_hbm, tab_b_hbm, idx_b_hbm,
                     out_a_hbm, out_b_hbm,
                     idx_v, rows_a, rows_b, sem):
    wid = lax.axis_index("s") * _NC + lax.axis_index("c")
    nk = idx_a_hbm.shape[0] * idx_a_hbm.shape[1]
    rows_per_w = nk // _NW
    chunks = rows_per_w // _CH
    crow0 = wid * chunks
    pltpu.sync_copy(idx_a_hbm.at[pl.ds(crow0, chunks)], idx_v)
    for c in range(chunks):
        base = (crow0 + c) * _CH
        pltpu.async_copy(tab_a_hbm.at[idx_v.at[c]], rows_a, sem).wait()
        pltpu.sync_copy(rows_a, out_a_hbm.at[pl.ds(base, _CH)])
    pltpu.sync_copy(idx_b_hbm.at[pl.ds(crow0, chunks)], idx_v)
    for c in range(chunks):
        base = (crow0 + c) * _CH
        pltpu.async_copy(tab_b_hbm.at[idx_v.at[c]], rows_b, sem).wait()
        pltpu.sync_copy(rows_b, out_b_hbm.at[pl.ds(base, _CH)])


def _sc_gather2(tab_a, idx_a, tab_b, idx_b):
    nk = idx_a.shape[0] * idx_a.shape[1]
    da, db = tab_a.shape[1], tab_b.shape[1]
    chunks = nk // _NW // _CH
    mesh = plsc.VectorSubcoreMesh(core_axis_name="c", subcore_axis_name="s")
    f = pl.kernel(
        _sc_gather2_body,
        out_type=[
            jax.ShapeDtypeStruct((nk, da), F32),
            jax.ShapeDtypeStruct((nk, db), F32),
        ],
        mesh=mesh,
        scratch_types=[
            pltpu.VMEM((chunks, _CH), jnp.int32),
            pltpu.VMEM((_CH, da), F32),
            pltpu.VMEM((_CH, db), F32),
            pltpu.SemaphoreType.DMA,
        ],
        compiler_params=pltpu.CompilerParams(use_tc_tiling_on_sc=False),
    )
    return f(tab_a, idx_a, tab_b, idx_b)


def _sc_gather1_body(tab_hbm, idx_hbm, out_hbm, idx_v, rows_v, sem):
    wid = lax.axis_index("s") * _NC + lax.axis_index("c")
    nk = idx_hbm.shape[0] * idx_hbm.shape[1]
    chunks = nk // _NW // _CH
    crow0 = wid * chunks
    pltpu.sync_copy(idx_hbm.at[pl.ds(crow0, chunks)], idx_v)
    for c in range(chunks):
        base = (crow0 + c) * _CH
        pltpu.async_copy(tab_hbm.at[idx_v.at[c]], rows_v, sem).wait()
        pltpu.sync_copy(rows_v, out_hbm.at[pl.ds(base, _CH)])


def _sc_gather1(tab, idx):
    nk = idx.shape[0] * idx.shape[1]
    d = tab.shape[1]
    chunks = nk // _NW // _CH
    mesh = plsc.VectorSubcoreMesh(core_axis_name="c", subcore_axis_name="s")
    f = pl.kernel(
        _sc_gather1_body,
        out_type=jax.ShapeDtypeStruct((nk, d), F32),
        mesh=mesh,
        scratch_types=[
            pltpu.VMEM((chunks, _CH), jnp.int32),
            pltpu.VMEM((_CH, d), F32),
            pltpu.SemaphoreType.DMA,
        ],
        compiler_params=pltpu.CompilerParams(use_tc_tiling_on_sc=False),
    )
    return f(tab, idx)


def _rep8(x):
    c = x.shape[1]
    return jnp.broadcast_to(x[:, None, :], (BQ, K, c)).reshape(GB, c)


def _sum8(x):
    c = x.shape[1]
    return x.reshape(BQ, K, c).sum(axis=1)


def _stage1_body(g_ref, t1_ref, x1_ref, wc_ref, b0_ref, m1_ref, b1_ref,
                 w0_ref, c0_ref, w1_ref, c1_ref, w2_ref, c2_ref, out_ref):
    g = g_ref[...]
    t1rep = _rep8(t1_ref[...])
    x1rep = _rep8(x1_ref[...])
    dxyz = g[:, D:D + 3] - x1rep
    h = g[:, 0:D] + t1rep + _dot(dxyz, wc_ref[...]) + b0_ref[...]
    h = _leaky(h)
    y = _leaky(_dot(h, m1_ref[...]) + b1_ref[...])
    w = _wnet(dxyz, w0_ref[...], c0_ref[...], w1_ref[...], c1_ref[...],
              w2_ref[...], c2_ref[...])
    prod = w * y
    out_ref[...] = _sum8(prod)


def _stage1(g, t1, x1r, wc, b0, m1w, b1, w0, c0, w1, c1, w2, c2):
    dg = g.shape[1]
    return pl.pallas_call(
        _stage1_body,
        grid=(N // BQ,),
        in_specs=[
            pl.BlockSpec((GB, dg), lambda i: (i, 0)),
            pl.BlockSpec((BQ, D), lambda i: (i, 0)),
            pl.BlockSpec((BQ, 3), lambda i: (i, 0)),
            pl.BlockSpec((3, D), lambda i: (0, 0)),
            pl.BlockSpec((1, D), lambda i: (0, 0)),
            pl.BlockSpec((D, 64), lambda i: (0, 0)),
            pl.BlockSpec((1, 64), lambda i: (0, 0)),
            pl.BlockSpec((3, 8), lambda i: (0, 0)),
            pl.BlockSpec((1, 8), lambda i: (0, 0)),
            pl.BlockSpec((8, 8), lambda i: (0, 0)),
            pl.BlockSpec((1, 8), lambda i: (0, 0)),
            pl.BlockSpec((8, 64), lambda i: (0, 0)),
            pl.BlockSpec((1, 64), lambda i: (0, 0)),
        ],
        out_specs=pl.BlockSpec((BQ, 64), lambda i: (i, 0)),
        out_shape=jax.ShapeDtypeStruct((N, 64), F32),
    )(g, t1, x1r, wc, b0, m1w, b1, w0, c0, w1, c1, w2, c2)


def _stage2_body(q_ref, g3_ref, x1_ref, w0_ref, c0_ref, w1_ref, c1_ref,
                 w2_ref, c2_ref, out_ref):
    x1rep = _rep8(x1_ref[...])
    dxyz2 = q_ref[...][:, 0:3] - x1rep
    w = _wnet(dxyz2, w0_ref[...], c0_ref[...], w1_ref[...], c1_ref[...],
              w2_ref[...], c2_ref[...])
    prod = w * g3_ref[...]
    out_ref[...] = _sum8(prod)


def _stage2(qg, g3, x1r, w0, c0, w1, c1, w2, c2):
    dq = qg.shape[1]
    return pl.pallas_call(
        _stage2_body,
        grid=(N // BQ,),
        in_specs=[
            pl.BlockSpec((GB, dq), lambda i: (i, 0)),
            pl.BlockSpec((GB, 64), lambda i: (i, 0)),
            pl.BlockSpec((BQ, 3), lambda i: (i, 0)),
            pl.BlockSpec((3, 8), lambda i: (0, 0)),
            pl.BlockSpec((1, 8), lambda i: (0, 0)),
            pl.BlockSpec((8, 8), lambda i: (0, 0)),
            pl.BlockSpec((1, 8), lambda i: (0, 0)),
            pl.BlockSpec((8, 64), lambda i: (0, 0)),
            pl.BlockSpec((1, 64), lambda i: (0, 0)),
        ],
        out_specs=pl.BlockSpec((BQ, 64), lambda i: (i, 0)),
        out_shape=jax.ShapeDtypeStruct((N, 64), F32),
    )(qg, g3, x1r, w0, c0, w1, c1, w2, c2)


def kernel(xyz1, xyz2, points1, points2, mlp0_w, mlp0_b, mlp1_w, mlp1_b,
           wn1_w0, wn1_b0, wn1_w1, wn1_b1, wn1_w2, wn1_b2,
           wn2_w0, wn2_b0, wn2_w1, wn2_b1, wn2_w2, wn2_b2):
    x1t = xyz1[0]
    x2t = xyz2[0]
    x1r = x1t.T
    x2r = x2t.T

    wa = mlp0_w[0:D]
    wb = mlp0_w[D:2 * D]
    wc = mlp0_w[2 * D:2 * D + 3]

    idx1, idx2 = _knn(x1r, x2t, x1t)
    t1, t2 = _proj(points1[0], points2[0], wa, wb)

    pad13 = jnp.zeros((N, 13), F32)
    tab2 = jnp.concatenate([t2, x2r, pad13], axis=1)
    tabq = jnp.concatenate([x1r, pad13], axis=1)

    idx1_2d = idx1.reshape(N * K // _CH, _CH)
    idx2_2d = idx2.reshape(N * K // _CH, _CH)

    g, qg = _sc_gather2(tab2, idx1_2d, tabq, idx2_2d)

    p2p = _stage1(g, t1, x1r, wc,
                  mlp0_b.reshape(1, D), mlp1_w, mlp1_b.reshape(1, 64),
                  wn1_w0, wn1_b0.reshape(1, 8), wn1_w1, wn1_b1.reshape(1, 8),
                  wn1_w2, wn1_b2.reshape(1, 64))

    g3 = _sc_gather1(p2p, idx2_2d)

    patch = _stage2(qg, g3, x1r,
                    wn2_w0, wn2_b0.reshape(1, 8), wn2_w1, wn2_b1.reshape(1, 8),
                    wn2_w2, wn2_b2.reshape(1, 64))

    return jnp.transpose(patch, (1, 0))[None]


# speedup vs baseline: 23.7785x; 1.0167x over previous
import functools

import jax
import jax.numpy as jnp
from jax import lax
from jax.experimental import pallas as pl
from jax.experimental.pallas import tpu as pltpu
from jax.experimental.pallas import tpu_sc as plsc

N = 8192
D = 128
K = 8
BQ = 128
BQK = 256
GB = BQ * K
F32 = jnp.float32


def _dot(a, b):
    return jnp.dot(a, b, preferred_element_type=F32,
                   precision=jax.lax.Precision.HIGHEST)


def _leaky(x):
    return jnp.where(x >= 0, x, 0.1 * x)


def _wnet(x, w0, b0, w1, b1, w2, b2):
    x = jax.nn.relu(_dot(x, w0) + b0)
    x = jax.nn.relu(_dot(x, w1) + b1)
    x = jax.nn.relu(_dot(x, w2) + b2)
    return x


def _top8(d, iota, out_cols):
    acc = jnp.zeros((BQK, K), jnp.int32)
    for s in range(K):
        m = jnp.min(d, axis=1, keepdims=True)
        eq = d <= m
        j = jnp.min(jnp.where(eq, iota, 0x7FFFFFFF), axis=1)
        acc = jnp.where(out_cols == s, j[:, None], acc)
        d = jnp.where(eq, 3e38, d)
    return acc


def _knn_body(q_ref, k2_ref, k1_ref, idx1_ref, idx2_ref):
    q = q_ref[...]
    k2 = k2_ref[...]
    k1 = k1_ref[...]
    qn = jnp.sum(q * q, axis=1, keepdims=True)
    iota = lax.broadcasted_iota(jnp.int32, (BQK, N), 1)
    out_cols = lax.broadcasted_iota(jnp.int32, (BQK, K), 1)

    kn2 = jnp.sum(k2 * k2, axis=0, keepdims=True)
    d1 = qn + kn2 - 2.0 * jnp.dot(q, k2, preferred_element_type=F32)
    idx1_ref[...] = _top8(d1, iota, out_cols)

    kn1 = jnp.sum(k1 * k1, axis=0, keepdims=True)
    d2 = qn + kn1 - 2.0 * jnp.dot(q, k1, preferred_element_type=F32)
    idx2_ref[...] = _top8(d2, iota, out_cols)


def _knn(x1r, x2t, x1t):
    return pl.pallas_call(
        _knn_body,
        grid=(N // BQK,),
        in_specs=[
            pl.BlockSpec((BQK, 3), lambda i: (i, 0)),
            pl.BlockSpec((3, N), lambda i: (0, 0)),
            pl.BlockSpec((3, N), lambda i: (0, 0)),
        ],
        out_specs=[
            pl.BlockSpec((BQK, K), lambda i: (i, 0)),
            pl.BlockSpec((BQK, K), lambda i: (i, 0)),
        ],
        out_shape=[
            jax.ShapeDtypeStruct((N, K), jnp.int32),
            jax.ShapeDtypeStruct((N, K), jnp.int32),
        ],
    )(x1r, x2t, x1t)


def _proj_body(p1_ref, p2_ref, wa_ref, wb_ref, t1_ref, t2_ref):
    dn = (((0,), (0,)), ((), ()))
    t1_ref[...] = lax.dot_general(p1_ref[...], wa_ref[...], dn,
                                  preferred_element_type=F32,
                                  precision=jax.lax.Precision.HIGHEST)
    t2_ref[...] = lax.dot_general(p2_ref[...], wb_ref[...], dn,
                                  preferred_element_type=F32,
                                  precision=jax.lax.Precision.HIGHEST)


def _proj(p1t, p2t, wa, wb):
    bm = 512
    return pl.pallas_call(
        _proj_body,
        grid=(N // bm,),
        in_specs=[
            pl.BlockSpec((D, bm), lambda i: (0, i)),
            pl.BlockSpec((D, bm), lambda i: (0, i)),
            pl.BlockSpec((D, D), lambda i: (0, 0)),
            pl.BlockSpec((D, D), lambda i: (0, 0)),
        ],
        out_specs=[
            pl.BlockSpec((bm, D), lambda i: (i, 0)),
            pl.BlockSpec((bm, D), lambda i: (i, 0)),
        ],
        out_shape=[
            jax.ShapeDtypeStruct((N, D), F32),
            jax.ShapeDtypeStruct((N, D), F32),
        ],
    )(p1t, p2t, wa, wb)


_NC = 2
_NS = 16
_NW = _NC * _NS
_CH = 128


def _sc_gather2_body(tab_a_hbm, idx_a_hbm, tab_b_hbm, idx_b_hbm,
                     out_a_hbm, out_b_hbm,
                     idx_v, rows_a, rows_b, sem):
    wid = lax.axis_index("s") * _NC + lax.axis_index("c")
    nk = idx_a_hbm.shape[0] * idx_a_hbm.shape[1]
    rows_per_w = nk // _NW
    chunks = rows_per_w // _CH
    crow0 = wid * chunks
    pltpu.sync_copy(idx_a_hbm.at[pl.ds(crow0, chunks)], idx_v)
    for c in range(chunks):
        base = (crow0 + c) * _CH
        pltpu.async_copy(tab_a_hbm.at[idx_v.at[c]], rows_a, sem).wait()
        pltpu.sync_copy(rows_a, out_a_hbm.at[pl.ds(base, _CH)])
    pltpu.sync_copy(idx_b_hbm.at[pl.ds(crow0, chunks)], idx_v)
    for c in range(chunks):
        base = (crow0 + c) * _CH
        pltpu.async_copy(tab_b_hbm.at[idx_v.at[c]], rows_b, sem).wait()
        pltpu.sync_copy(rows_b, out_b_hbm.at[pl.ds(base, _CH)])


def _sc_gather2(tab_a, idx_a, tab_b, idx_b):
    nk = idx_a.shape[0] * idx_a.shape[1]
    da, db = tab_a.shape[1], tab_b.shape[1]
    chunks = nk // _NW // _CH
    mesh = plsc.VectorSubcoreMesh(core_axis_name="c", subcore_axis_name="s")
    f = pl.kernel(
        _sc_gather2_body,
        out_type=[
            jax.ShapeDtypeStruct((nk, da), F32),
            jax.ShapeDtypeStruct((nk, db), F32),
        ],
        mesh=mesh,
        scratch_types=[
            pltpu.VMEM((chunks, _CH), jnp.int32),
            pltpu.VMEM((_CH, da), F32),
            pltpu.VMEM((_CH, db), F32),
            pltpu.SemaphoreType.DMA,
        ],
        compiler_params=pltpu.CompilerParams(use_tc_tiling_on_sc=False),
    )
    return f(tab_a, idx_a, tab_b, idx_b)


def _sc_gather1_body(tab_hbm, idx_hbm, out_hbm, idx_v, rows_v, sem):
    wid = lax.axis_index("s") * _NC + lax.axis_index("c")
    nk = idx_hbm.shape[0] * idx_hbm.shape[1]
    chunks = nk // _NW // _CH
    crow0 = wid * chunks
    pltpu.sync_copy(idx_hbm.at[pl.ds(crow0, chunks)], idx_v)
    for c in range(chunks):
        base = (crow0 + c) * _CH
        pltpu.async_copy(tab_hbm.at[idx_v.at[c]], rows_v, sem).wait()
        pltpu.sync_copy(rows_v, out_hbm.at[pl.ds(base, _CH)])


def _sc_gather1(tab, idx):
    nk = idx.shape[0] * idx.shape[1]
    d = tab.shape[1]
    chunks = nk // _NW // _CH
    mesh = plsc.VectorSubcoreMesh(core_axis_name="c", subcore_axis_name="s")
    f = pl.kernel(
        _sc_gather1_body,
        out_type=jax.ShapeDtypeStruct((nk, d), F32),
        mesh=mesh,
        scratch_types=[
            pltpu.VMEM((chunks, _CH), jnp.int32),
            pltpu.VMEM((_CH, d), F32),
            pltpu.SemaphoreType.DMA,
        ],
        compiler_params=pltpu.CompilerParams(use_tc_tiling_on_sc=False),
    )
    return f(tab, idx)


def _rep8(x):
    c = x.shape[1]
    return jnp.broadcast_to(x[:, None, :], (BQ, K, c)).reshape(GB, c)


def _sum8(x):
    c = x.shape[1]
    return x.reshape(BQ, K, c).sum(axis=1)


def _stage1_body(g_ref, t1_ref, x1_ref, wc_ref, b0_ref, m1_ref, b1_ref,
                 w0_ref, c0_ref, w1_ref, c1_ref, w2_ref, c2_ref, out_ref):
    g = g_ref[...]
    t1rep = _rep8(t1_ref[...])
    x1rep = _rep8(x1_ref[...])
    dxyz = g[:, D:D + 3] - x1rep
    h = g[:, 0:D] + t1rep + _dot(dxyz, wc_ref[...]) + b0_ref[...]
    h = _leaky(h)
    y = _leaky(_dot(h, m1_ref[...]) + b1_ref[...])
    w = _wnet(dxyz, w0_ref[...], c0_ref[...], w1_ref[...], c1_ref[...],
              w2_ref[...], c2_ref[...])
    prod = w * y
    out_ref[...] = _sum8(prod)


def _stage1(g, t1, x1r, wc, b0, m1w, b1, w0, c0, w1, c1, w2, c2):
    dg = g.shape[1]
    return pl.pallas_call(
        _stage1_body,
        grid=(N // BQ,),
        in_specs=[
            pl.BlockSpec((GB, dg), lambda i: (i, 0)),
            pl.BlockSpec((BQ, D), lambda i: (i, 0)),
            pl.BlockSpec((BQ, 3), lambda i: (i, 0)),
            pl.BlockSpec((3, D), lambda i: (0, 0)),
            pl.BlockSpec((1, D), lambda i: (0, 0)),
            pl.BlockSpec((D, 64), lambda i: (0, 0)),
            pl.BlockSpec((1, 64), lambda i: (0, 0)),
            pl.BlockSpec((3, 8), lambda i: (0, 0)),
            pl.BlockSpec((1, 8), lambda i: (0, 0)),
            pl.BlockSpec((8, 8), lambda i: (0, 0)),
            pl.BlockSpec((1, 8), lambda i: (0, 0)),
            pl.BlockSpec((8, 64), lambda i: (0, 0)),
            pl.BlockSpec((1, 64), lambda i: (0, 0)),
        ],
        out_specs=pl.BlockSpec((BQ, 64), lambda i: (i, 0)),
        out_shape=jax.ShapeDtypeStruct((N, 64), F32),
    )(g, t1, x1r, wc, b0, m1w, b1, w0, c0, w1, c1, w2, c2)


def _stage2_body(q_ref, g3_ref, x1_ref, w0_ref, c0_ref, w1_ref, c1_ref,
                 w2_ref, c2_ref, out_ref):
    x1rep = _rep8(x1_ref[...])
    dxyz2 = q_ref[...][:, 0:3] - x1rep
    w = _wnet(dxyz2, w0_ref[...], c0_ref[...], w1_ref[...], c1_ref[...],
              w2_ref[...], c2_ref[...])
    prod = w * g3_ref[...]
    out_ref[...] = _sum8(prod).T


def _stage2(qg, g3, x1r, w0, c0, w1, c1, w2, c2):
    dq = qg.shape[1]
    return pl.pallas_call(
        _stage2_body,
        grid=(N // BQ,),
        in_specs=[
            pl.BlockSpec((GB, dq), lambda i: (i, 0)),
            pl.BlockSpec((GB, 64), lambda i: (i, 0)),
            pl.BlockSpec((BQ, 3), lambda i: (i, 0)),
            pl.BlockSpec((3, 8), lambda i: (0, 0)),
            pl.BlockSpec((1, 8), lambda i: (0, 0)),
            pl.BlockSpec((8, 8), lambda i: (0, 0)),
            pl.BlockSpec((1, 8), lambda i: (0, 0)),
            pl.BlockSpec((8, 64), lambda i: (0, 0)),
            pl.BlockSpec((1, 64), lambda i: (0, 0)),
        ],
        out_specs=pl.BlockSpec((64, BQ), lambda i: (0, i)),
        out_shape=jax.ShapeDtypeStruct((64, N), F32),
    )(qg, g3, x1r, w0, c0, w1, c1, w2, c2)


def kernel(xyz1, xyz2, points1, points2, mlp0_w, mlp0_b, mlp1_w, mlp1_b,
           wn1_w0, wn1_b0, wn1_w1, wn1_b1, wn1_w2, wn1_b2,
           wn2_w0, wn2_b0, wn2_w1, wn2_b1, wn2_w2, wn2_b2):
    x1t = xyz1[0]
    x2t = xyz2[0]
    x1r = x1t.T
    x2r = x2t.T

    wa = mlp0_w[0:D]
    wb = mlp0_w[D:2 * D]
    wc = mlp0_w[2 * D:2 * D + 3]

    idx1, idx2 = _knn(x1r, x2t, x1t)
    t1, t2 = _proj(points1[0], points2[0], wa, wb)

    pad13 = jnp.zeros((N, 13), F32)
    tab2 = jnp.concatenate([t2, x2r, pad13], axis=1)
    tabq = jnp.concatenate([x1r, pad13], axis=1)

    idx1_2d = idx1.reshape(N * K // _CH, _CH)
    idx2_2d = idx2.reshape(N * K // _CH, _CH)

    g, qg = _sc_gather2(tab2, idx1_2d, tabq, idx2_2d)

    p2p = _stage1(g, t1, x1r, wc,
                  mlp0_b.reshape(1, D), mlp1_w, mlp1_b.reshape(1, 64),
                  wn1_w0, wn1_b0.reshape(1, 8), wn1_w1, wn1_b1.reshape(1, 8),
                  wn1_w2, wn1_b2.reshape(1, 64))

    g3 = _sc_gather1(p2p, idx2_2d)

    patch = _stage2(qg, g3, x1r,
                    wn2_w0, wn2_b0.reshape(1, 8), wn2_w1, wn2_b1.reshape(1, 8),
                    wn2_w2, wn2_b2.reshape(1, 64))

    return patch[None]


# speedup vs baseline: 24.0946x; 1.0133x over previous
import functools

import jax
import jax.numpy as jnp
from jax import lax
from jax.experimental import pallas as pl
from jax.experimental.pallas import tpu as pltpu
from jax.experimental.pallas import tpu_sc as plsc

N = 8192
D = 128
K = 8
BQ = 128
BQK = 256
GB = BQ * K
F32 = jnp.float32


def _dot(a, b):
    return jnp.dot(a, b, preferred_element_type=F32,
                   precision=jax.lax.Precision.HIGHEST)


def _leaky(x):
    return jnp.where(x >= 0, x, 0.1 * x)


def _wnet(x, w0, b0, w1, b1, w2, b2):
    x = jax.nn.relu(_dot(x, w0) + b0)
    x = jax.nn.relu(_dot(x, w1) + b1)
    x = jax.nn.relu(_dot(x, w2) + b2)
    return x


def _top8(d, iota, out_cols):
    acc = jnp.zeros((BQK, K), jnp.int32)
    for s in range(K):
        m = jnp.min(d, axis=1, keepdims=True)
        eq = d <= m
        j = jnp.min(jnp.where(eq, iota, 0x7FFFFFFF), axis=1)
        acc = jnp.where(out_cols == s, j[:, None], acc)
        d = jnp.where(eq, 3e38, d)
    return acc


def _knn_body(q_ref, k2_ref, k1_ref, idx1_ref, idx2_ref):
    q = q_ref[...]
    k2 = k2_ref[...]
    k1 = k1_ref[...]
    qn = jnp.sum(q * q, axis=1, keepdims=True)
    iota = lax.broadcasted_iota(jnp.int32, (BQK, N), 1)
    out_cols = lax.broadcasted_iota(jnp.int32, (BQK, K), 1)

    kn2 = jnp.sum(k2 * k2, axis=0, keepdims=True)
    d1 = qn + kn2 - 2.0 * jnp.dot(q, k2, preferred_element_type=F32)
    idx1_ref[...] = _top8(d1, iota, out_cols)

    kn1 = jnp.sum(k1 * k1, axis=0, keepdims=True)
    d2 = qn + kn1 - 2.0 * jnp.dot(q, k1, preferred_element_type=F32)
    idx2_ref[...] = _top8(d2, iota, out_cols)


def _knn(x1r, x2t, x1t):
    return pl.pallas_call(
        _knn_body,
        grid=(N // BQK,),
        in_specs=[
            pl.BlockSpec((BQK, 3), lambda i: (i, 0)),
            pl.BlockSpec((3, N), lambda i: (0, 0)),
            pl.BlockSpec((3, N), lambda i: (0, 0)),
        ],
        out_specs=[
            pl.BlockSpec((BQK, K), lambda i: (i, 0)),
            pl.BlockSpec((BQK, K), lambda i: (i, 0)),
        ],
        out_shape=[
            jax.ShapeDtypeStruct((N, K), jnp.int32),
            jax.ShapeDtypeStruct((N, K), jnp.int32),
        ],
    )(x1r, x2t, x1t)


def _proj_body(p1_ref, p2_ref, wa_ref, wb_ref, t1_ref, t2_ref):
    dn = (((0,), (0,)), ((), ()))
    t1_ref[...] = lax.dot_general(p1_ref[...], wa_ref[...], dn,
                                  preferred_element_type=F32,
                                  precision=jax.lax.Precision.HIGHEST)
    t2_ref[...] = lax.dot_general(p2_ref[...], wb_ref[...], dn,
                                  preferred_element_type=F32,
                                  precision=jax.lax.Precision.HIGHEST)


def _proj(p1t, p2t, wa, wb):
    bm = 512
    return pl.pallas_call(
        _proj_body,
        grid=(N // bm,),
        in_specs=[
            pl.BlockSpec((D, bm), lambda i: (0, i)),
            pl.BlockSpec((D, bm), lambda i: (0, i)),
            pl.BlockSpec((D, D), lambda i: (0, 0)),
            pl.BlockSpec((D, D), lambda i: (0, 0)),
        ],
        out_specs=[
            pl.BlockSpec((bm, D), lambda i: (i, 0)),
            pl.BlockSpec((bm, D), lambda i: (i, 0)),
        ],
        out_shape=[
            jax.ShapeDtypeStruct((N, D), F32),
            jax.ShapeDtypeStruct((N, D), F32),
        ],
    )(p1t, p2t, wa, wb)


_NC = 2
_NS = 16
_NW = _NC * _NS
_CH = 128


def _gather_chunks_2buf(tab_hbm, out_hbm, idx_v, buf0, buf1, sem0, sem1,
                        crow0, chunks):
    bufs = (buf0, buf1)
    sems = (sem0, sem1)
    pend = pltpu.async_copy(tab_hbm.at[idx_v.at[0]], bufs[0], sems[0])
    for c in range(chunks):
        cur = c & 1
        if c + 1 < chunks:
            nxt = pltpu.async_copy(tab_hbm.at[idx_v.at[c + 1]],
                                   bufs[1 - cur], sems[1 - cur])
        pend.wait()
        pltpu.sync_copy(bufs[cur], out_hbm.at[pl.ds((crow0 + c) * _CH, _CH)])
        if c + 1 < chunks:
            pend = nxt


def _sc_gather2_body(tab_a_hbm, idx_a_hbm, tab_b_hbm, idx_b_hbm,
                     out_a_hbm, out_b_hbm,
                     idx_v, rows_a0, rows_a1, rows_b0, rows_b1,
                     sem0, sem1):
    wid = lax.axis_index("s") * _NC + lax.axis_index("c")
    nk = idx_a_hbm.shape[0] * idx_a_hbm.shape[1]
    rows_per_w = nk // _NW
    chunks = rows_per_w // _CH
    crow0 = wid * chunks
    pltpu.sync_copy(idx_a_hbm.at[pl.ds(crow0, chunks)], idx_v)
    _gather_chunks_2buf(tab_a_hbm, out_a_hbm, idx_v, rows_a0, rows_a1,
                        sem0, sem1, crow0, chunks)
    pltpu.sync_copy(idx_b_hbm.at[pl.ds(crow0, chunks)], idx_v)
    _gather_chunks_2buf(tab_b_hbm, out_b_hbm, idx_v, rows_b0, rows_b1,
                        sem0, sem1, crow0, chunks)


def _sc_gather2(tab_a, idx_a, tab_b, idx_b):
    nk = idx_a.shape[0] * idx_a.shape[1]
    da, db = tab_a.shape[1], tab_b.shape[1]
    chunks = nk // _NW // _CH
    mesh = plsc.VectorSubcoreMesh(core_axis_name="c", subcore_axis_name="s")
    f = pl.kernel(
        _sc_gather2_body,
        out_type=[
            jax.ShapeDtypeStruct((nk, da), F32),
            jax.ShapeDtypeStruct((nk, db), F32),
        ],
        mesh=mesh,
        scratch_types=[
            pltpu.VMEM((chunks, _CH), jnp.int32),
            pltpu.VMEM((_CH, da), F32),
            pltpu.VMEM((_CH, da), F32),
            pltpu.VMEM((_CH, db), F32),
            pltpu.VMEM((_CH, db), F32),
            pltpu.SemaphoreType.DMA,
            pltpu.SemaphoreType.DMA,
        ],
        compiler_params=pltpu.CompilerParams(use_tc_tiling_on_sc=False),
    )
    return f(tab_a, idx_a, tab_b, idx_b)


def _sc_gather1_body(tab_hbm, idx_hbm, out_hbm, idx_v, rows0, rows1,
                     sem0, sem1):
    wid = lax.axis_index("s") * _NC + lax.axis_index("c")
    nk = idx_hbm.shape[0] * idx_hbm.shape[1]
    chunks = nk // _NW // _CH
    crow0 = wid * chunks
    pltpu.sync_copy(idx_hbm.at[pl.ds(crow0, chunks)], idx_v)
    _gather_chunks_2buf(tab_hbm, out_hbm, idx_v, rows0, rows1,
                        sem0, sem1, crow0, chunks)


def _sc_gather1(tab, idx):
    nk = idx.shape[0] * idx.shape[1]
    d = tab.shape[1]
    chunks = nk // _NW // _CH
    mesh = plsc.VectorSubcoreMesh(core_axis_name="c", subcore_axis_name="s")
    f = pl.kernel(
        _sc_gather1_body,
        out_type=jax.ShapeDtypeStruct((nk, d), F32),
        mesh=mesh,
        scratch_types=[
            pltpu.VMEM((chunks, _CH), jnp.int32),
            pltpu.VMEM((_CH, d), F32),
            pltpu.VMEM((_CH, d), F32),
            pltpu.SemaphoreType.DMA,
            pltpu.SemaphoreType.DMA,
        ],
        compiler_params=pltpu.CompilerParams(use_tc_tiling_on_sc=False),
    )
    return f(tab, idx)


def _rep8(x):
    c = x.shape[1]
    return jnp.broadcast_to(x[:, None, :], (BQ, K, c)).reshape(GB, c)


def _sum8(x):
    c = x.shape[1]
    return x.reshape(BQ, K, c).sum(axis=1)


def _stage1_body(g_ref, t1_ref, x1_ref, wc_ref, b0_ref, m1_ref, b1_ref,
                 w0_ref, c0_ref, w1_ref, c1_ref, w2_ref, c2_ref, out_ref):
    g = g_ref[...]
    t1rep = _rep8(t1_ref[...])
    x1rep = _rep8(x1_ref[...])
    dxyz = g[:, D:D + 3] - x1rep
    h = g[:, 0:D] + t1rep + _dot(dxyz, wc_ref[...]) + b0_ref[...]
    h = _leaky(h)
    y = _leaky(_dot(h, m1_ref[...]) + b1_ref[...])
    w = _wnet(dxyz, w0_ref[...], c0_ref[...], w1_ref[...], c1_ref[...],
              w2_ref[...], c2_ref[...])
    prod = w * y
    out_ref[...] = _sum8(prod)


def _stage1(g, t1, x1r, wc, b0, m1w, b1, w0, c0, w1, c1, w2, c2):
    dg = g.shape[1]
    return pl.pallas_call(
        _stage1_body,
        grid=(N // BQ,),
        in_specs=[
            pl.BlockSpec((GB, dg), lambda i: (i, 0)),
            pl.BlockSpec((BQ, D), lambda i: (i, 0)),
            pl.BlockSpec((BQ, 3), lambda i: (i, 0)),
            pl.BlockSpec((3, D), lambda i: (0, 0)),
            pl.BlockSpec((1, D), lambda i: (0, 0)),
            pl.BlockSpec((D, 64), lambda i: (0, 0)),
            pl.BlockSpec((1, 64), lambda i: (0, 0)),
            pl.BlockSpec((3, 8), lambda i: (0, 0)),
            pl.BlockSpec((1, 8), lambda i: (0, 0)),
            pl.BlockSpec((8, 8), lambda i: (0, 0)),
            pl.BlockSpec((1, 8), lambda i: (0, 0)),
            pl.BlockSpec((8, 64), lambda i: (0, 0)),
            pl.BlockSpec((1, 64), lambda i: (0, 0)),
        ],
        out_specs=pl.BlockSpec((BQ, 64), lambda i: (i, 0)),
        out_shape=jax.ShapeDtypeStruct((N, 64), F32),
    )(g, t1, x1r, wc, b0, m1w, b1, w0, c0, w1, c1, w2, c2)


def _stage2_body(q_ref, g3_ref, x1_ref, w0_ref, c0_ref, w1_ref, c1_ref,
                 w2_ref, c2_ref, out_ref):
    x1rep = _rep8(x1_ref[...])
    dxyz2 = q_ref[...][:, 0:3] - x1rep
    w = _wnet(dxyz2, w0_ref[...], c0_ref[...], w1_ref[...], c1_ref[...],
              w2_ref[...], c2_ref[...])
    prod = w * g3_ref[...]
    out_ref[...] = _sum8(prod).T


def _stage2(qg, g3, x1r, w0, c0, w1, c1, w2, c2):
    dq = qg.shape[1]
    return pl.pallas_call(
        _stage2_body,
        grid=(N // BQ,),
        in_specs=[
            pl.BlockSpec((GB, dq), lambda i: (i, 0)),
            pl.BlockSpec((GB, 64), lambda i: (i, 0)),
            pl.BlockSpec((BQ, 3), lambda i: (i, 0)),
            pl.BlockSpec((3, 8), lambda i: (0, 0)),
            pl.BlockSpec((1, 8), lambda i: (0, 0)),
            pl.BlockSpec((8, 8), lambda i: (0, 0)),
            pl.BlockSpec((1, 8), lambda i: (0, 0)),
            pl.BlockSpec((8, 64), lambda i: (0, 0)),
            pl.BlockSpec((1, 64), lambda i: (0, 0)),
        ],
        out_specs=pl.BlockSpec((64, BQ), lambda i: (0, i)),
        out_shape=jax.ShapeDtypeStruct((64, N), F32),
    )(qg, g3, x1r, w0, c0, w1, c1, w2, c2)


def kernel(xyz1, xyz2, points1, points2, mlp0_w, mlp0_b, mlp1_w, mlp1_b,
           wn1_w0, wn1_b0, wn1_w1, wn1_b1, wn1_w2, wn1_b2,
           wn2_w0, wn2_b0, wn2_w1, wn2_b1, wn2_w2, wn2_b2):
    x1t = xyz1[0]
    x2t = xyz2[0]
    x1r = x1t.T
    x2r = x2t.T

    wa = mlp0_w[0:D]
    wb = mlp0_w[D:2 * D]
    wc = mlp0_w[2 * D:2 * D + 3]

    idx1, idx2 = _knn(x1r, x2t, x1t)
    t1, t2 = _proj(points1[0], points2[0], wa, wb)

    pad13 = jnp.zeros((N, 13), F32)
    tab2 = jnp.concatenate([t2, x2r, pad13], axis=1)
    tabq = jnp.concatenate([x1r, pad13], axis=1)

    idx1_2d = idx1.reshape(N * K // _CH, _CH)
    idx2_2d = idx2.reshape(N * K // _CH, _CH)

    g, qg = _sc_gather2(tab2, idx1_2d, tabq, idx2_2d)

    p2p = _stage1(g, t1, x1r, wc,
                  mlp0_b.reshape(1, D), mlp1_w, mlp1_b.reshape(1, 64),
                  wn1_w0, wn1_b0.reshape(1, 8), wn1_w1, wn1_b1.reshape(1, 8),
                  wn1_w2, wn1_b2.reshape(1, 64))

    g3 = _sc_gather1(p2p, idx2_2d)

    patch = _stage2(qg, g3, x1r,
                    wn2_w0, wn2_b0.reshape(1, 8), wn2_w1, wn2_b1.reshape(1, 8),
                    wn2_w2, wn2_b2.reshape(1, 64))

    return patch[None]
